# Initial kernel scaffold; baseline (speedup 1.0000x reference)
import jax
import jax.numpy as jnp
from jax.experimental import pallas as pl


def kernel(x, edge_index, W_gcn, b_gcn, W_sage_l, b_sage_l, W_sage_r, W_gcn1, b_gcn1, W_gr_rel, b_gr, W_gr_root, W_out, b_out):
    raise NotImplementedError("write your pallas kernel here")


# speedup vs baseline: 5.9270x; 5.9270x over previous
import functools

import jax
import jax.numpy as jnp
from jax import lax
from jax.experimental import pallas as pl
from jax.experimental.pallas import tpu as pltpu
from jax.experimental.pallas import tpu_sc as plsc

NC = 2
NS = 16
LANE = 128


def _cdiv(a, b):
  return (a + b - 1) // b


def _degree_kernel(n_acc, dch):
  mesh = plsc.VectorSubcoreMesh(core_axis_name="c", subcore_axis_name="s",
                                num_cores=NC, num_subcores=NS)
  rows_per_tile = n_acc // NS
  nz = rows_per_tile // 128

  @functools.partial(
      pl.kernel, mesh=mesh,
      out_type=jax.ShapeDtypeStruct((NC * n_acc, LANE), jnp.float32),
      scratch_types=[
          pltpu.VMEM((dch, LANE), jnp.int32),
          pltpu.VMEM((128, LANE), jnp.float32),
          pltpu.VMEM_SHARED((n_acc, LANE), jnp.float32),
      ],
  )
  def k(didx_hbm, ones_hbm, zeros_hbm, out_hbm, didx_v, ones_v, acc):
    c = lax.axis_index("c")
    s = lax.axis_index("s")
    tid = c * NS + s
    pltpu.sync_copy(didx_hbm.at[pl.ds(tid * dch, dch)], didx_v)
    pltpu.sync_copy(zeros_hbm, ones_v)
    for z in range(nz):
      pltpu.sync_copy(ones_v,
                      acc.at[pl.ds(s * rows_per_tile + z * 128, 128)])
    plsc.subcore_barrier()
    pltpu.sync_copy(ones_hbm, ones_v)

    for j in range(dch):
      pltpu.sync_copy(ones_v, acc.at[didx_v.at[j]], add=True)

    plsc.subcore_barrier()
    pltpu.sync_copy(acc.at[pl.ds(s * rows_per_tile, rows_per_tile)],
                    out_hbm.at[pl.ds(c * n_acc + s * rows_per_tile,
                                     rows_per_tile)])

  return k


_PIECE = 16


def _agg_kernel(n_acc, ch):
  mesh = plsc.VectorSubcoreMesh(core_axis_name="c", subcore_axis_name="s",
                                num_cores=NC, num_subcores=NS)
  rpt = n_acc // NS
  nz = rpt // 128
  np_ = ch // _PIECE
  rows = ch + _PIECE

  @functools.partial(
      pl.kernel, mesh=mesh,
      out_type=jax.ShapeDtypeStruct((NC * n_acc, LANE), jnp.float32),
      scratch_types=[
          pltpu.VMEM((_PIECE, LANE), jnp.int32),
          pltpu.VMEM((_PIECE, LANE), jnp.int32),
          pltpu.VMEM((_PIECE, LANE), jnp.int32),
          pltpu.VMEM((_PIECE, LANE), jnp.int32),
          pltpu.VMEM((128, LANE), jnp.float32),
          pltpu.VMEM((128, LANE), jnp.float32),
          pltpu.VMEM_SHARED((n_acc, LANE), jnp.float32),
          pltpu.SemaphoreType.DMA,
          pltpu.SemaphoreType.DMA,
          pltpu.SemaphoreType.DMA,
          pltpu.SemaphoreType.DMA,
      ],
  )
  def k(tbl_hbm, gidx_hbm, didx_hbm, zeros_hbm, out_hbm,
        sp0, sp1, dp0, dp1, buf0, buf1, acc, gsem0, gsem1, isem_s, isem_d):
    c = lax.axis_index("c")
    s = lax.axis_index("s")
    gbase = (c * NS + s) * rows
    dbase = s * rows
    sp = (sp0, sp1)
    dp = (dp0, dp1)
    buf = (buf0, buf1)
    gsem = (gsem0, gsem1)

    def spiece(g, pbuf):
      return pltpu.make_async_copy(
          gidx_hbm.at[pl.ds(gbase + g * _PIECE, _PIECE)], pbuf, isem_s)

    def dpiece(g, pbuf):
      return pltpu.make_async_copy(
          didx_hbm.at[pl.ds(dbase + g * _PIECE, _PIECE)], pbuf, isem_d)

    def gather(pbuf, row, b):
      return pltpu.make_async_copy(tbl_hbm.at[pbuf.at[row]], buf[b], gsem[b])

    pltpu.sync_copy(zeros_hbm, buf0)
    for z in range(nz):
      pltpu.sync_copy(buf0, acc.at[pl.ds(s * rpt + z * 128, 128)])
    plsc.subcore_barrier()

    spiece(0, sp0).start()
    dpiece(0, dp0).start()
    pltpu.make_async_copy(gidx_hbm.at[pl.ds(gbase, _PIECE)], sp0,
                          isem_s).wait()
    pltpu.make_async_copy(didx_hbm.at[pl.ds(dbase, _PIECE)], dp0,
                          isem_d).wait()
    gather(sp0, 0, 0).start()
    gather(sp0, 1, 1).start()

    for g in range(np_):
      pg = g % 2
      pn = (g + 1) % 2
      sp_next = spiece(g + 1, sp[pn])
      dp_next = dpiece(g + 1, dp[pn])
      sp_next.start()
      dp_next.start()
      for ci in range(_PIECE):
        b = ci % 2
        if ci == _PIECE - 2:
          sp_next.wait()
          dp_next.wait()
        gather(sp[pg], ci, b).wait()
        pltpu.sync_copy(buf[b], acc.at[dp[pg].at[ci]], add=True)
        if ci < _PIECE - 2:
          gather(sp[pg], ci + 2, b).start()
        else:
          gather(sp[pn], ci + 2 - _PIECE, b).start()

    gather(sp0, 0, 0).wait()
    gather(sp0, 1, 1).wait()

    plsc.subcore_barrier()
    pltpu.sync_copy(acc.at[pl.ds(s * rpt, rpt)],
                    out_hbm.at[pl.ds(c * n_acc + s * rpt, rpt)])

  return k


_BR = 400


def _rows(br=None):
  br = _BR if br is None else br
  return pl.BlockSpec((br, LANE), lambda i: (i, 0))


def _col():
  return pl.BlockSpec((_BR, 1), lambda i: (i, 0))


def _full(r, c=LANE):
  return pl.BlockSpec((r, c), lambda i: (0, 0))


def _stage1_body(cnt0_ref, cnt1_ref, x_ref, y0_ref, cnt_ref):
  cnt = cnt0_ref[...] + cnt1_ref[...]
  dinv = lax.rsqrt(cnt + 1.0)
  y0_ref[...] = dinv * x_ref[...]
  cnt_ref[...] = cnt


def _stage2_body(x_ref, ay_ref, ax_ref, cnt_ref,
                 wg_ref, bg_ref, wl_ref, bl_ref, wr_ref,
                 x1_ref, y1_ref, x2_ref):
  cnt = cnt_ref[...]
  dinv = lax.rsqrt(cnt + 1.0)
  x = x_ref[...]
  z1 = dinv * ay_ref[...] + (dinv * dinv) * x
  x1 = jnp.maximum(
      jnp.dot(z1, wg_ref[...], preferred_element_type=jnp.float32)
      + bg_ref[...], 0.0)
  x1_ref[...] = x1
  y1_ref[...] = dinv * x1
  mean = ax_ref[...] / jnp.maximum(cnt, 1.0)
  x2_ref[...] = jnp.maximum(
      jnp.dot(mean, wl_ref[...], preferred_element_type=jnp.float32)
      + jnp.dot(x, wr_ref[...], preferred_element_type=jnp.float32)
      + bl_ref[...], 0.0)


def _stage3_body(x1_ref, ay1_ref, ax1_ref, cnt_ref, x2_ref,
                 wg1_ref, bg1_ref, wrel_ref, bgr_ref, wroot_ref,
                 wo1_ref, wo2_ref, wo3_ref, bo_ref, out_ref):
  cnt = cnt_ref[...]
  dinv = lax.rsqrt(cnt + 1.0)
  x1 = x1_ref[...]
  z3 = dinv * ay1_ref[...] + (dinv * dinv) * x1
  x3 = jnp.maximum(
      jnp.dot(z3, wg1_ref[...], preferred_element_type=jnp.float32)
      + bg1_ref[...], 0.0)
  x4 = jnp.maximum(
      jnp.dot(ax1_ref[...], wrel_ref[...], preferred_element_type=jnp.float32)
      + jnp.dot(x1, wroot_ref[...], preferred_element_type=jnp.float32)
      + bgr_ref[...], 0.0)
  logits = (jnp.dot(x2_ref[...], wo1_ref[...],
                    preferred_element_type=jnp.float32)
            + jnp.dot(x3, wo2_ref[...], preferred_element_type=jnp.float32)
            + jnp.dot(x4, wo3_ref[...], preferred_element_type=jnp.float32)
            + bo_ref[...])
  out_ref[...] = jax.nn.sigmoid(logits)


def kernel(x, edge_index, W_gcn, b_gcn, W_sage_l, b_sage_l, W_sage_r,
           W_gcn1, b_gcn1, W_gr_rel, b_gr, W_gr_root, W_out, b_out):
  n, d = x.shape
  e = edge_index.shape[1]
  f32 = jnp.float32

  src = edge_index[0].astype(jnp.int32)
  dst = edge_index[1].astype(jnp.int32)

  n_acc = _cdiv(n + 1, NS * 128) * NS * 128
  scrap = jnp.int32(n)

  dch = _cdiv(e, NC * NS * 128)
  dch = _cdiv(dch, 8) * 8
  e_deg = NC * NS * dch * 128
  dst_deg = jnp.concatenate(
      [dst, jnp.full((e_deg - e,), scrap, jnp.int32)]).reshape(-1, 128)

  ch = _cdiv(e, NS * 128)
  ch = _cdiv(ch, _PIECE) * _PIECE
  e_agg = NS * ch * 128
  pad_e = e_agg - e
  src_a = jnp.concatenate([src, jnp.zeros((pad_e,), jnp.int32)])
  dst_a = jnp.concatenate([dst, jnp.full((pad_e,), scrap, jnp.int32)])
  src_t = jnp.concatenate(
      [src_a.reshape(NS, ch, 128), jnp.zeros((NS, _PIECE, 128), jnp.int32)],
      axis=1)
  gidx = jnp.concatenate([src_t, src_t + n], axis=0).reshape(-1, 128)
  didx = jnp.concatenate(
      [dst_a.reshape(NS, ch, 128),
       jnp.full((NS, _PIECE, 128), scrap, jnp.int32)], axis=1).reshape(-1, 128)

  ones128 = jnp.ones((128, LANE), f32)
  zeros128 = jnp.zeros((128, LANE), f32)

  deg_out = _degree_kernel(n_acc, dch)(dst_deg, ones128, zeros128)
  cnt0 = deg_out[:n, :1]
  cnt1 = deg_out[n_acc:n_acc + n, :1]

  grid = (_cdiv(n, _BR),)
  y0, cnt = pl.pallas_call(
      _stage1_body,
      grid=grid,
      in_specs=[_col(), _col(), _rows()],
      out_specs=[_rows(), _col()],
      out_shape=[jax.ShapeDtypeStruct((n, d), f32),
                 jax.ShapeDtypeStruct((n, 1), f32)],
  )(cnt0, cnt1, x)

  agg = _agg_kernel(n_acc, ch)
  tbl_a = jnp.concatenate([y0, x], axis=0)
  out_a = agg(tbl_a, gidx, didx, zeros128)
  agg_y0 = out_a[:n]
  agg_x = out_a[n_acc:n_acc + n]

  b_gcn2 = b_gcn.reshape(1, -1)
  b_sage2 = b_sage_l.reshape(1, -1)
  x1, y1, x2 = pl.pallas_call(
      _stage2_body,
      grid=grid,
      in_specs=[_rows(), _rows(), _rows(), _col(),
                _full(d), _full(1), _full(d), _full(1), _full(d)],
      out_specs=[_rows(), _rows(), _rows()],
      out_shape=[jax.ShapeDtypeStruct((n, d), f32)] * 3,
  )(x, agg_y0, agg_x, cnt, W_gcn, b_gcn2, W_sage_l, b_sage2, W_sage_r)

  tbl_b = jnp.concatenate([y1, x1], axis=0)
  out_b = agg(tbl_b, gidx, didx, zeros128)
  agg_y1 = out_b[:n]
  agg_x1 = out_b[n_acc:n_acc + n]

  wo1 = W_out[:d]
  wo2 = W_out[d:2 * d]
  wo3 = W_out[2 * d:]
  out = pl.pallas_call(
      _stage3_body,
      grid=grid,
      in_specs=[_rows(), _rows(), _rows(), _col(), _rows(),
                _full(d), _full(1), _full(d), _full(1), _full(d),
                _full(d), _full(d), _full(d), _full(1)],
      out_specs=_rows(),
      out_shape=jax.ShapeDtypeStruct((n, d), f32),
  )(x1, agg_y1, agg_x1, cnt, x2,
    W_gcn1, b_gcn1.reshape(1, -1), W_gr_rel, b_gr.reshape(1, -1), W_gr_root,
    wo1, wo2, wo3, b_out.reshape(1, -1))
  return out
